# baseline (speedup 1.0000x reference)
import functools

import jax
import jax.numpy as jnp
from jax import lax
from jax.experimental import pallas as pl
from jax.experimental.pallas import tpu as pltpu
from jax.experimental.pallas import tpu_sc as plsc

_V = 1000000
_D = 64
_H = 50
_B = 4096
_NCLS = 100
_NC = 2
_NS = 16
_NW = _NC * _NS
_SPW = _B // _NW
_CH = 8
_ROWS = _CH * _H
_NCHUNK = _SPW // _CH
_NSTEP = 2 * _NCHUNK
_VBLK = 8192
_VHALF = _VBLK // 2
_NBLK = (_V + _VBLK - 1) // _VBLK
_VROWS = _NBLK * _VHALF


def _transform(emb_A, emb_B, W1):
    a_t = emb_A.T
    b_t = emb_B.T

    def body(a_ref, b_ref, w_ref, u_ref):
        w = w_ref[...] * (1.0 / _H)
        qi = jax.lax.broadcasted_iota(jnp.int32, (2 * _D, 2 * _D), 0)
        qj = jax.lax.broadcasted_iota(jnp.int32, (2 * _D, 2 * _D), 1)
        keep = (qi < _D) == (qj < _D)
        w_bd = jnp.where(
            keep,
            jnp.concatenate([w, w], axis=1),
            0.0,
        )
        lhs = jnp.concatenate([a_ref[...], b_ref[...]], axis=0)
        dn = (((0,), (0,)), ((), ()))
        u = lax.dot_general(lhs, w_bd, dn, preferred_element_type=jnp.float32)
        bits = lax.bitcast_convert_type(u, jnp.int32)
        pa = bits[:, :_D] & jnp.int32(-65536)
        pb = lax.shift_right_logical(bits[:, _D:], 16)
        packed = pa | pb
        u_ref[...] = jnp.concatenate(
            [packed[:_VHALF], packed[_VHALF:]], axis=1)

    return pl.pallas_call(
        body,
        grid=(_NBLK,),
        in_specs=[
            pl.BlockSpec((_D, _VBLK), lambda i: (0, i)),
            pl.BlockSpec((_D, _VBLK), lambda i: (0, i)),
            pl.BlockSpec((2 * _D, _D), lambda i: (0, 0)),
        ],
        out_specs=pl.BlockSpec((_VHALF, 2 * _D), lambda i: (i, 0)),
        out_shape=jax.ShapeDtypeStruct((_VROWS, 2 * _D), jnp.int32),
        compiler_params=pltpu.CompilerParams(
            fuse_transposed_lhs_in_matmul=True),
    )(a_t, b_t, W1)


def _sc_pool(ids_a, ids_b, table):
    mesh = plsc.VectorSubcoreMesh(
        core_axis_name="c", subcore_axis_name="s",
        num_cores=_NC, num_subcores=_NS)

    @functools.partial(
        pl.kernel,
        out_type=jax.ShapeDtypeStruct((_B, _D), jnp.float32),
        mesh=mesh,
        scratch_types=[
            pltpu.VMEM((_ROWS + 16,), jnp.int32),
            pltpu.VMEM((_ROWS + 16,), jnp.int32),
            pltpu.VMEM((_ROWS,), jnp.int32),
            pltpu.VMEM((_ROWS,), jnp.int32),
            pltpu.VMEM((_ROWS, 2 * _D), jnp.int32),
            pltpu.VMEM((_ROWS, 2 * _D), jnp.int32),
            pltpu.VMEM((_SPW, _D), jnp.float32),
            pltpu.SemaphoreType.DMA,
            pltpu.SemaphoreType.DMA,
            pltpu.SemaphoreType.DMA,
            pltpu.SemaphoreType.DMA,
        ],
    )
    def pool(idsA_hbm, idsB_hbm, tab_hbm, out_hbm,
             idx0_v, idx1_v, gdx0_v, gdx1_v, rows0_v, rows1_v, feat_v,
             isem0, isem1, rsem0, rsem1):
        idxs = (idx0_v, idx1_v)
        gdxs = (gdx0_v, gdx1_v)
        rows = (rows0_v, rows1_v)
        isems = (isem0, isem1)
        rsems = (rsem0, rsem1)
        wid = lax.axis_index("s") * _NC + lax.axis_index("c")
        sbase = wid * _SPW

        def start_idx(b, step):
            t, c = divmod(step, _NCHUNK)
            ids_hbm = idsA_hbm if t == 0 else idsB_hbm
            off = sbase * _H + c * _ROWS
            return pltpu.async_copy(
                ids_hbm.at[pl.ds(off, _ROWS)],
                idxs[b].at[pl.ds(0, _ROWS)], isems[b])

        def compute_gidx(b):
            for k in range(_ROWS // 16):
                v = idxs[b][pl.ds(k * 16, 16)]
                gdxs[b][pl.ds(k * 16, 16)] = (
                    lax.shift_left(lax.shift_right_logical(v, 13), 12)
                    | (v & 4095))

        def start_gather(b, step):
            return pltpu.async_copy(
                tab_hbm.at[gdxs[b]], rows[b], rsems[b])

        def reduce_chunk(b, step):
            t, c = divmod(step, _NCHUNK)

            def body(r, accs):
                out = []
                for s in range(_CH):
                    row = s * _H + r
                    iv = idxs[b][pl.ds(row, 16)]
                    sel = (iv[0] & 4096) != 0
                    for q in range(_D // 16):
                        v_lo = rows[b][row, pl.ds(q * 16, 16)]
                        v_hi = rows[b][row, pl.ds(_D + q * 16, 16)]
                        v = jnp.where(sel, v_hi, v_lo)
                        if t == 0:
                            f = lax.bitcast_convert_type(
                                v & jnp.int32(-65536), jnp.float32)
                        else:
                            f = lax.bitcast_convert_type(
                                lax.shift_left(v, 16), jnp.float32)
                        out.append(accs[s * (_D // 16) + q] + f)
                return tuple(out)

            init = tuple(jnp.zeros((16,), jnp.float32)
                         for _ in range(_CH * (_D // 16)))
            accs = lax.fori_loop(0, _H, body, init)
            for s in range(_CH):
                for q in range(_D // 16):
                    sl = pl.ds(q * 16, 16)
                    a = accs[s * (_D // 16) + q]
                    if t == 0:
                        feat_v[c * _CH + s, sl] = a
                    else:
                        feat_v[c * _CH + s, sl] = feat_v[c * _CH + s, sl] + a

        h_idx = [None, None]
        h_row = [None, None]
        h_idx[0] = start_idx(0, 0)
        h_idx[0].wait()
        compute_gidx(0)
        h_row[0] = start_gather(0, 0)
        h_idx[1] = start_idx(1, 1)
        for step in range(_NSTEP):
            b = step % 2
            h_row[b].wait()
            if step + 1 < _NSTEP:
                h_idx[1 - b].wait()
                compute_gidx(1 - b)
                h_row[1 - b] = start_gather(1 - b, step + 1)
            reduce_chunk(b, step)
            if step + 2 < _NSTEP:
                h_idx[b] = start_idx(b, step + 2)
        pltpu.sync_copy(feat_v, out_hbm.at[pl.ds(sbase, _SPW), :])

    return pool(ids_a, ids_b, table)


def _mlp(s, b1, W2, b2):
    def body(s_ref, b1_ref, w2_ref, b2_ref, o_ref):
        h = jnp.maximum(s_ref[...] + b1_ref[...], 0.0)
        o_ref[...] = (jnp.dot(h, w2_ref[...],
                              preferred_element_type=jnp.float32)
                      + b2_ref[...])

    return pl.pallas_call(
        body,
        out_shape=jax.ShapeDtypeStruct((_B, _NCLS), jnp.float32),
    )(s, b1.reshape(1, _D), W2, b2.reshape(1, _NCLS))


def kernel(ids_A, ids_B, emb_A, emb_B, W1, b1, W2, b2):
    table = _transform(emb_A, emb_B, W1)
    s = _sc_pool(ids_A.astype(jnp.int32).reshape(-1),
                 ids_B.astype(jnp.int32).reshape(-1),
                 table)
    return _mlp(s, b1, W2, b2)


# speedup vs baseline: 1.0717x; 1.0717x over previous
import functools

import jax
import jax.numpy as jnp
from jax import lax
from jax.experimental import pallas as pl
from jax.experimental.pallas import tpu as pltpu
from jax.experimental.pallas import tpu_sc as plsc

_V = 1000000
_D = 64
_H = 50
_B = 4096
_NCLS = 100
_NC = 2
_NS = 16
_NW = _NC * _NS
_SPW = _B // _NW
_CH = 8
_ROWS = _CH * _H
_NCHUNK = _SPW // _CH
_NSTEP = 2 * _NCHUNK
_VBLK = 16384


def _transform(emb_A, emb_B, W1):
    a_t = emb_A.T
    b_t = emb_B.T
    grid = (_V + _VBLK - 1) // _VBLK

    def body(a_ref, b_ref, w_ref, u_ref):
        w = w_ref[...] * (1.0 / _H)
        qi = jax.lax.broadcasted_iota(jnp.int32, (2 * _D, 2 * _D), 0)
        qj = jax.lax.broadcasted_iota(jnp.int32, (2 * _D, 2 * _D), 1)
        keep = (qi < _D) == (qj < _D)
        w_bd = jnp.where(
            keep,
            jnp.concatenate([w, w], axis=1),
            0.0,
        )
        lhs = jnp.concatenate([a_ref[...], b_ref[...]], axis=0)
        dn = (((0,), (0,)), ((), ()))
        u_ref[...] = lax.dot_general(
            lhs, w_bd, dn, preferred_element_type=jnp.float32)

    return pl.pallas_call(
        body,
        grid=(grid,),
        in_specs=[
            pl.BlockSpec((_D, _VBLK), lambda i: (0, i)),
            pl.BlockSpec((_D, _VBLK), lambda i: (0, i)),
            pl.BlockSpec((2 * _D, _D), lambda i: (0, 0)),
        ],
        out_specs=pl.BlockSpec((_VBLK, 2 * _D), lambda i: (i, 0)),
        out_shape=jax.ShapeDtypeStruct((_V, 2 * _D), jnp.float32),
        compiler_params=pltpu.CompilerParams(
            fuse_transposed_lhs_in_matmul=True),
    )(a_t, b_t, W1)


def _sc_pool(ids_a, ids_b, table):
    mesh = plsc.VectorSubcoreMesh(
        core_axis_name="c", subcore_axis_name="s",
        num_cores=_NC, num_subcores=_NS)

    @functools.partial(
        pl.kernel,
        out_type=jax.ShapeDtypeStruct((_B, _D), jnp.float32),
        mesh=mesh,
        scratch_types=[
            pltpu.VMEM((_ROWS,), jnp.int32),
            pltpu.VMEM((_ROWS,), jnp.int32),
            pltpu.VMEM((_ROWS, 2 * _D), jnp.float32),
            pltpu.VMEM((_ROWS, 2 * _D), jnp.float32),
            pltpu.VMEM((_SPW, _D), jnp.float32),
            pltpu.SemaphoreType.DMA,
            pltpu.SemaphoreType.DMA,
            pltpu.SemaphoreType.DMA,
            pltpu.SemaphoreType.DMA,
        ],
    )
    def pool(idsA_hbm, idsB_hbm, tab_hbm, out_hbm,
             idx0_v, idx1_v, rows0_v, rows1_v, feat_v,
             isem0, isem1, rsem0, rsem1):
        idxs = (idx0_v, idx1_v)
        rows = (rows0_v, rows1_v)
        isems = (isem0, isem1)
        rsems = (rsem0, rsem1)
        wid = lax.axis_index("s") * _NC + lax.axis_index("c")
        sbase = wid * _SPW

        def start_idx(b, step):
            t, c = divmod(step, _NCHUNK)
            ids_hbm = idsA_hbm if t == 0 else idsB_hbm
            off = sbase * _H + c * _ROWS
            return pltpu.async_copy(
                ids_hbm.at[pl.ds(off, _ROWS)], idxs[b], isems[b])

        def start_gather(b, step):
            return pltpu.async_copy(
                tab_hbm.at[idxs[b]], rows[b], rsems[b])

        def reduce_chunk(b, step):
            t, c = divmod(step, _NCHUNK)
            col0 = t * _D

            def body(r, accs):
                out = []
                for s in range(_CH):
                    for q in range(_D // 16):
                        v = rows[b][s * _H + r, pl.ds(col0 + q * 16, 16)]
                        out.append(accs[s * (_D // 16) + q] + v)
                return tuple(out)

            init = tuple(jnp.zeros((16,), jnp.float32)
                         for _ in range(_CH * (_D // 16)))
            accs = lax.fori_loop(0, _H, body, init)
            for s in range(_CH):
                for q in range(_D // 16):
                    sl = pl.ds(q * 16, 16)
                    a = accs[s * (_D // 16) + q]
                    if t == 0:
                        feat_v[c * _CH + s, sl] = a
                    else:
                        feat_v[c * _CH + s, sl] = feat_v[c * _CH + s, sl] + a

        h_idx = [None, None]
        h_row = [None, None]
        h_idx[0] = start_idx(0, 0)
        h_idx[0].wait()
        h_row[0] = start_gather(0, 0)
        h_idx[1] = start_idx(1, 1)
        for step in range(_NSTEP):
            b = step % 2
            h_row[b].wait()
            if step + 2 < _NSTEP:
                h_idx[b] = start_idx(b, step + 2)
            if step + 1 < _NSTEP:
                h_idx[1 - b].wait()
                h_row[1 - b] = start_gather(1 - b, step + 1)
            reduce_chunk(b, step)
        pltpu.sync_copy(feat_v, out_hbm.at[pl.ds(sbase, _SPW), :])

    return pool(ids_a, ids_b, table)


def _mlp(s, b1, W2, b2):
    def body(s_ref, b1_ref, w2_ref, b2_ref, o_ref):
        h = jnp.maximum(s_ref[...] + b1_ref[...], 0.0)
        o_ref[...] = (jnp.dot(h, w2_ref[...],
                              preferred_element_type=jnp.float32)
                      + b2_ref[...])

    return pl.pallas_call(
        body,
        out_shape=jax.ShapeDtypeStruct((_B, _NCLS), jnp.float32),
    )(s, b1.reshape(1, _D), W2, b2.reshape(1, _NCLS))


def kernel(ids_A, ids_B, emb_A, emb_B, W1, b1, W2, b2):
    table = _transform(emb_A, emb_B, W1)
    s = _sc_pool(ids_A.astype(jnp.int32).reshape(-1),
                 ids_B.astype(jnp.int32).reshape(-1),
                 table)
    return _mlp(s, b1, W2, b2)
